# Initial kernel scaffold; baseline (speedup 1.0000x reference)
import jax
import jax.numpy as jnp
from jax.experimental import pallas as pl


def kernel(x):
    raise NotImplementedError("write your pallas kernel here")


# speedup vs baseline: 27.6117x; 27.6117x over previous
import functools

import jax
import jax.numpy as jnp
from jax.experimental import pallas as pl
from jax.experimental.pallas import tpu as pltpu

_K = 819


def _topk_mask_norm_kernel(x_ref, o_ref):
    x = x_ref[...]
    bits = jax.lax.bitcast_convert_type(jnp.abs(x), jnp.int32)

    def step(i, piv):
        b = 30 - i
        trial = piv | (1 << b)
        cnt = jnp.sum((bits >= trial).astype(jnp.int32), axis=1, keepdims=True)
        return jnp.where(cnt >= _K, trial, piv)

    piv = jax.lax.fori_loop(0, 31, step, jnp.zeros((64, 1), jnp.int32))
    m = (bits >= piv).astype(jnp.float32)
    union = jnp.max(m, axis=0, keepdims=True)
    y = x * union
    s = jnp.sum(y * y, axis=1, keepdims=True)
    o_ref[...] = y / (jnp.sqrt(s) + 1e-6)


@jax.jit
def kernel(x):
    return pl.pallas_call(
        _topk_mask_norm_kernel,
        out_shape=jax.ShapeDtypeStruct(x.shape, x.dtype),
    )(x)
